# baseline (speedup 1.0000x reference)
import jax
import jax.numpy as jnp
from jax import lax
from jax.experimental import pallas as pl
from jax.experimental.pallas import tpu as pltpu

_MAX_MASK_LEN = 20
_WIN = 256
_NBK = 6
_NWN = 4


def _mask_params(B, D):
    key = jax.random.key(42)
    k1, k2 = jax.random.split(key)
    hi = min(_MAX_MASK_LEN, D // 4)
    mask_len = jax.random.randint(k1, (1,), 1, hi)
    ml = mask_len[0]
    mask_start = jax.random.randint(k2, (B,), 0, jnp.maximum(1, D - ml))
    return ml, mask_start


def _make_body(B, T, D):
    ncase = (D - _WIN) // 128 + 1

    def body(s_ref, x_ref, o_ref, bbuf, wbuf, bsi, bso, wsi, wso):
        ml = s_ref[0]

        def wlo_of(b):
            return jnp.minimum(s_ref[1 + b] // 128, (D - _WIN) // 128)

        def bulk(b, inward):
            slot = b % _NBK
            sem = bsi.at[slot] if inward else bso.at[slot]
            wlo = wlo_of(b)
            for k in range(ncase):

                @pl.when(wlo == k)
                def _(b=b, k=k):
                    lw = k * 128
                    if lw > 0:
                        h = x_ref.at[b, :, pl.ds(0, lw)]
                        v = bbuf.at[b % _NBK, :, pl.ds(0, lw)]
                        d = o_ref.at[b, :, pl.ds(0, lw)]
                        c = (pltpu.make_async_copy(h, v, sem) if inward
                             else pltpu.make_async_copy(v, d, sem))
                        c.start()
                    if lw + _WIN < D:
                        off = lw + _WIN
                        h = x_ref.at[b, :, pl.ds(off, D - off)]
                        v = bbuf.at[b % _NBK, :, pl.ds(lw, D - off)]
                        d = o_ref.at[b, :, pl.ds(off, D - off)]
                        c = (pltpu.make_async_copy(h, v, sem) if inward
                             else pltpu.make_async_copy(v, d, sem))
                        c.start()

        def bulk_wait(b, inward):
            slot = b % _NBK
            sem = bsi.at[slot] if inward else bso.at[slot]
            pltpu.make_async_copy(
                x_ref.at[b, :, pl.ds(0, _WIN)], bbuf.at[slot], sem).wait()

        def win_gather(b):
            off = pl.multiple_of(wlo_of(b) * 128, 128)
            return pltpu.make_async_copy(
                x_ref.at[b, :, pl.ds(off, _WIN)], wbuf.at[b % _NWN],
                wsi.at[b % _NWN])

        def win_scatter(b):
            off = pl.multiple_of(wlo_of(b) * 128, 128)
            return pltpu.make_async_copy(
                wbuf.at[b % _NWN], o_ref.at[b, :, pl.ds(off, _WIN)],
                wso.at[b % _NWN])

        for b in range(min(_NBK - 1, B)):
            bulk(b, inward=True)
        for b in range(min(_NWN - 1, B)):
            win_gather(b).start()

        iota = lax.broadcasted_iota(jnp.int32, (1, 128), 1)
        for b in range(B):
            i = b % _NWN
            start = s_ref[1 + b]
            wlo = wlo_of(b)
            c0 = start // 128
            c1 = (start + ml - 1) // 128

            win_gather(b).wait()

            def _fix(cb):
                p = pl.multiple_of((cb - wlo) * 128, 128)
                col = cb * 128 + iota
                mask = (col >= start) & (col < start + ml)
                cur = wbuf[i, :, pl.ds(p, 128)]
                wbuf[i, :, pl.ds(p, 128)] = jnp.where(
                    mask, jnp.float32(0.0), cur)

            _fix(c0)

            @pl.when(c1 != c0)
            def _():
                _fix(c1)

            win_scatter(b).start()

            bulk_wait(b, inward=True)
            bulk(b, inward=False)

            nb = b + _NBK - 1
            if nb < B:
                if nb >= _NBK:
                    bulk_wait(nb - _NBK, inward=False)
                bulk(nb, inward=True)
            nw = b + _NWN - 1
            if nw < B:
                if nw >= _NWN:
                    win_scatter(nw - _NWN).wait()
                win_gather(nw).start()

        for b in range(max(B - _NBK, 0), B):
            bulk_wait(b, inward=False)
        for b in range(max(B - _NWN, 0), B):
            win_scatter(b).wait()

    return body


def kernel(mean):
    B, T, D = mean.shape
    ml, mask_start = _mask_params(B, D)
    scalars = jnp.concatenate([ml[None], mask_start]).astype(jnp.int32)

    return pl.pallas_call(
        _make_body(B, T, D),
        in_specs=[
            pl.BlockSpec(memory_space=pltpu.SMEM),
            pl.BlockSpec(memory_space=pl.ANY),
        ],
        out_specs=pl.BlockSpec(memory_space=pl.ANY),
        out_shape=jax.ShapeDtypeStruct((B, T, D), mean.dtype),
        scratch_shapes=[
            pltpu.VMEM((_NBK, T, _WIN), jnp.float32),
            pltpu.VMEM((_NWN, T, _WIN), jnp.float32),
            pltpu.SemaphoreType.DMA((_NBK,)),
            pltpu.SemaphoreType.DMA((_NBK,)),
            pltpu.SemaphoreType.DMA((_NWN,)),
            pltpu.SemaphoreType.DMA((_NWN,)),
        ],
    )(scalars, mean)
